# Initial kernel scaffold; baseline (speedup 1.0000x reference)
import jax
import jax.numpy as jnp
from jax.experimental import pallas as pl


def kernel(x, edge_index, params):
    raise NotImplementedError("write your pallas kernel here")


# speedup vs baseline: 2.4990x; 2.4990x over previous
import functools

import jax
import jax.numpy as jnp
from jax import lax
from jax.experimental import pallas as pl
from jax.experimental.pallas import tpu as pltpu
from jax.experimental.pallas import tpu_sc as plsc

N = 50000
E = 800000
IN_DIM = 19
H = 64
HH = 32

NC = 2
NS = 16
CH = 128
CHUNKS_PER_TILE = 391
EPAD = NS * CH * CHUNKS_PER_TILE
ACC_ROWS = 51200
DUMP_ROW = N
WB = 80
WB_TOTAL = N // WB
WB_PER_TILE = (WB_TOTAL + NS - 1) // NS
ZCH = 128
ZCHUNKS = ACC_ROWS // (NS * ZCH)

BLK = 2000
GRID = N // BLK


def _sc_agg_body(table, g0, s0, g1, s1, zrows, out,
                 gidx, gadj, sidx, rows, zbuf, wbuf, sem, acc):
    cid = lax.axis_index("c")
    sid = lax.axis_index("s")
    off = (cid * N).astype(jnp.int32)
    edge_base = sid * (CHUNKS_PER_TILE * CH)

    pltpu.sync_copy(zrows, zbuf)

    def zero_acc(acc):
        zbase = sid * (ZCHUNKS * ZCH)

        def zloop(z, _):
            pltpu.sync_copy(zbuf, acc.at[pl.ds(zbase + z * ZCH, ZCH)])
            return _

        lax.fori_loop(0, ZCHUNKS, zloop, None)

    def accumulate(acc, g_hbm, s_hbm):
        def eloop(c, _):
            base = edge_base + c * CH
            pltpu.sync_copy(g_hbm.at[pl.ds(base, CH)], gidx)
            for k in range(CH // 16):
                gadj[pl.ds(k * 16, 16)] = gidx[pl.ds(k * 16, 16)] + off
            pltpu.async_copy(table.at[gadj], rows, sem).wait()
            pltpu.sync_copy(s_hbm.at[pl.ds(base, CH)], sidx)
            pltpu.sync_copy(rows, acc.at[sidx], add=True)
            return _

        lax.fori_loop(0, CHUNKS_PER_TILE, eloop, None)

    def writeback(acc, out_base):
        def wloop(w, _):
            k = sid + NS * w

            @pl.when(k < WB_TOTAL)
            def _do():
                r = k * WB
                pltpu.sync_copy(acc.at[pl.ds(r, WB)], wbuf)
                pltpu.sync_copy(wbuf,
                                out.at[pl.ds(out_base + cid * N + r, WB)])

            return _

        lax.fori_loop(0, WB_PER_TILE, wloop, None)

    zero_acc(acc)
    plsc.subcore_barrier()
    accumulate(acc, g0, s0)
    plsc.subcore_barrier()
    writeback(acc, 0)
    plsc.subcore_barrier()
    zero_acc(acc)
    plsc.subcore_barrier()
    accumulate(acc, g1, s1)
    plsc.subcore_barrier()
    writeback(acc, 2 * N)


@functools.cache
def _sc_agg():
    return pl.kernel(
        _sc_agg_body,
        out_type=jax.ShapeDtypeStruct((4 * N, HH), jnp.float32),
        mesh=plsc.VectorSubcoreMesh(core_axis_name="c", subcore_axis_name="s",
                                    num_cores=NC),
        compiler_params=pltpu.CompilerParams(use_tc_tiling_on_sc=False),
        scratch_types=[
            pltpu.VMEM((CH,), jnp.int32),
            pltpu.VMEM((CH,), jnp.int32),
            pltpu.VMEM((CH,), jnp.int32),
            pltpu.VMEM((CH, HH), jnp.float32),
            pltpu.VMEM((ZCH, HH), jnp.float32),
            pltpu.VMEM((WB, HH), jnp.float32),
            pltpu.SemaphoreType.DMA,
            pltpu.VMEM_SHARED((ACC_ROWS, HH), jnp.float32),
        ],
    )


def _ln(h, g, b):
    mu = jnp.mean(h, axis=-1, keepdims=True)
    var = jnp.mean((h - mu) ** 2, axis=-1, keepdims=True)
    return (h - mu) / jnp.sqrt(var + 1e-5) * g + b


def _mlp(h, W1, g, b, W2):
    h = jnp.dot(h, W1, preferred_element_type=jnp.float32, precision=jax.lax.Precision.HIGHEST)
    h = jax.nn.relu(_ln(h, g, b))
    return jnp.dot(h, W2, preferred_element_type=jnp.float32, precision=jax.lax.Precision.HIGHEST)


def _mlp_split_in(h, a0, a1, scale, W1, g, b, W2):
    t = scale * jnp.dot(h, W1, preferred_element_type=jnp.float32, precision=jax.lax.Precision.HIGHEST)
    t += jnp.dot(a0, W1[:HH, :], preferred_element_type=jnp.float32, precision=jax.lax.Precision.HIGHEST)
    t += jnp.dot(a1, W1[HH:, :], preferred_element_type=jnp.float32, precision=jax.lax.Precision.HIGHEST)
    t = jax.nn.relu(_ln(t, g, b))
    return jnp.dot(t, W2, preferred_element_type=jnp.float32, precision=jax.lax.Precision.HIGHEST)


def _downup_dense(h, a, p):
    (dW1, dg, db, dW2, deps, uW1, ug, ub, uW2, ueps,
     ln1g, ln1b, ln2g, ln2b, dire, cW, cb) = p
    hd = _mlp_split_in(h, a[0], a[1], 1.0 + deps[0, 0], dW1, dg, db, dW2)
    hd = _ln(jax.nn.relu(hd + dire[0:1, :]), ln1g, ln1b)
    hu = _mlp_split_in(h, a[2], a[3], 1.0 + ueps[0, 0], uW1, ug, ub, uW2)
    hu = _ln(jax.nn.relu(hu + dire[1:2, :]), ln2g, ln2b)
    out = jnp.dot(hd, cW[:H, :], preferred_element_type=jnp.float32, precision=jax.lax.Precision.HIGHEST)
    out += jnp.dot(hu, cW[H:, :], preferred_element_type=jnp.float32, precision=jax.lax.Precision.HIGHEST)
    return out + cb


def _kernel_a_body(x_ref, W1, g, b, W2, h_ref, tab_ref):
    h = _mlp(x_ref[...], W1[...], g[...], b[...], W2[...])
    h_ref[...] = h
    tab_ref[0] = h[:, :HH]
    tab_ref[1] = h[:, HH:]


def _kernel_b_body(h_ref, a_ref, dW1, dg, db, dW2, deps, uW1, ug, ub, uW2,
                   ueps, ln1g, ln1b, ln2g, ln2b, dire, cW, cb,
                   eW1, eg, eb, eW2, h2_ref, tab2_ref):
    p = (dW1[...], dg[...], db[...], dW2[...], deps[...], uW1[...], ug[...],
         ub[...], uW2[...], ueps[...], ln1g[...], ln1b[...], ln2g[...],
         ln2b[...], dire[...], cW[...], cb[...])
    h1 = _downup_dense(h_ref[...], a_ref[...], p)
    h2 = _mlp(h1, eW1[...], eg[...], eb[...], eW2[...])
    h2_ref[...] = h2
    tab2_ref[0] = h2[:, :HH]
    tab2_ref[1] = h2[:, HH:]


def _kernel_c_body(h_ref, a_ref, dW1, dg, db, dW2, deps, uW1, ug, ub, uW2,
                   ueps, ln1g, ln1b, ln2g, ln2b, dire, cW, cb,
                   oW1, og, ob, oW2, out_ref):
    p = (dW1[...], dg[...], db[...], dW2[...], deps[...], uW1[...], ug[...],
         ub[...], uW2[...], ueps[...], ln1g[...], ln1b[...], ln2g[...],
         ln2b[...], dire[...], cW[...], cb[...])
    h3 = _downup_dense(h_ref[...], a_ref[...], p)
    x_raw = _mlp(h3, oW1[...], og[...], ob[...], oW2[...])
    blk = x_raw.shape[0]
    ci = lax.broadcasted_iota(jnp.int32, (blk, IN_DIM), 1)
    m_axis = (ci >= 7) & (ci < 10)
    s = jnp.sum(jnp.where(m_axis, x_raw * x_raw, 0.0), axis=-1, keepdims=True)
    norm = jnp.maximum(jnp.sqrt(s), 1e-6)
    out = jnp.where(ci < 3, jax.nn.sigmoid(x_raw),
                    jnp.where(m_axis, x_raw / norm, x_raw))
    out_ref[...] = out


def _whole(shape):
    return pl.BlockSpec(shape, lambda i: (0,) * len(shape))


def _rows(width):
    return pl.BlockSpec((BLK, width), lambda i: (i, 0))


_A_SPECS = dict(
    grid=(GRID,),
    in_specs=[_rows(IN_DIM), _whole((IN_DIM, H)), _whole((1, H)),
              _whole((1, H)), _whole((H, H))],
    out_specs=[_rows(H), pl.BlockSpec((2, BLK, HH), lambda i: (0, i, 0))],
)

_DU_PARAM_SPECS = [
    _whole((H, H)), _whole((1, H)), _whole((1, H)), _whole((H, H)),
    _whole((1, 1)),
    _whole((H, H)), _whole((1, H)), _whole((1, H)), _whole((H, H)),
    _whole((1, 1)),
    _whole((1, H)), _whole((1, H)), _whole((1, H)), _whole((1, H)),
    _whole((2, H)), _whole((2 * H, H)), _whole((1, H)),
]

_B_SPECS = dict(
    grid=(GRID,),
    in_specs=[_rows(H), pl.BlockSpec((4, BLK, HH), lambda i: (0, i, 0))]
             + _DU_PARAM_SPECS
             + [_whole((H, H)), _whole((1, H)), _whole((1, H)),
                _whole((H, H))],
    out_specs=[_rows(H), pl.BlockSpec((2, BLK, HH), lambda i: (0, i, 0))],
)

_C_SPECS = dict(
    grid=(GRID,),
    in_specs=[_rows(H), pl.BlockSpec((4, BLK, HH), lambda i: (0, i, 0))]
             + _DU_PARAM_SPECS
             + [_whole((H, H)), _whole((1, H)), _whole((1, H)),
                _whole((H, IN_DIM))],
    out_specs=_rows(IN_DIM),
)


def _mlp_args(p):
    return (p['W1'], p['g'].reshape(1, -1), p['b'].reshape(1, -1), p['W2'])


def _du_args(p):
    return (*_mlp_args(p['down']), p['down_eps'].reshape(1, 1),
            *_mlp_args(p['up']), p['up_eps'].reshape(1, 1),
            p['ln1_g'].reshape(1, -1), p['ln1_b'].reshape(1, -1),
            p['ln2_g'].reshape(1, -1), p['ln2_b'].reshape(1, -1),
            p['dir_emb'], p['combine_W'], p['combine_b'].reshape(1, -1))


def kernel(x, edge_index, params):
    src = edge_index[0]
    dst = edge_index[1]
    npad = EPAD - E
    zpad = jnp.zeros((npad,), jnp.int32)
    dpad = jnp.full((npad,), DUMP_ROW, jnp.int32)
    src_g = jnp.concatenate([src, zpad])
    src_s = jnp.concatenate([src, dpad])
    dst_g = jnp.concatenate([dst, zpad])
    dst_s = jnp.concatenate([dst, dpad])
    zrows = jnp.zeros((ZCH, HH), jnp.float32)

    h0, tab0 = pl.pallas_call(
        _kernel_a_body,
        out_shape=[jax.ShapeDtypeStruct((N, H), jnp.float32),
                   jax.ShapeDtypeStruct((2, N, HH), jnp.float32)],
        **_A_SPECS,
    )(x, *_mlp_args(params['enc_in']))

    agg0 = _sc_agg()(tab0.reshape(2 * N, HH), src_g, dst_s, dst_g, src_s,
                     zrows)

    h2, tab2 = pl.pallas_call(
        _kernel_b_body,
        out_shape=[jax.ShapeDtypeStruct((N, H), jnp.float32),
                   jax.ShapeDtypeStruct((2, N, HH), jnp.float32)],
        **_B_SPECS,
    )(h0, agg0.reshape(4, N, HH), *_du_args(params['enc_layer']),
      *_mlp_args(params['dec_in']))

    agg2 = _sc_agg()(tab2.reshape(2 * N, HH), src_g, dst_s, dst_g, src_s,
                     zrows)

    x_hat = pl.pallas_call(
        _kernel_c_body,
        out_shape=jax.ShapeDtypeStruct((N, IN_DIM), jnp.float32),
        **_C_SPECS,
    )(h2, agg2.reshape(4, N, HH), *_du_args(params['dec_layer']),
      *_mlp_args(params['out_proj']))

    return x_hat
